# Initial kernel scaffold; baseline (speedup 1.0000x reference)
import jax
import jax.numpy as jnp
from jax.experimental import pallas as pl


def kernel(atomic_numbers, edge_index, edge_dist, batch, emb_table, proj_W, proj_b, edge_W, edge_b, Wm, bm, Wu, bu, out_W1, out_b1, out_W2, out_b2):
    raise NotImplementedError("write your pallas kernel here")


# speedup vs baseline: 3.1624x; 3.1624x over previous
import functools

import jax
import jax.numpy as jnp
import numpy as np
from jax import lax
from jax.experimental import pallas as pl
from jax.experimental.pallas import tpu as pltpu
from jax.experimental.pallas import tpu_sc as plsc

_MASSES = [0.0, 1.008, 4.003, 6.941, 9.012, 10.81, 12.01, 14.01, 16.0, 19.0, 20.18, 22.99, 24.31, 26.98, 28.09, 30.97, 32.07, 35.45, 39.95, 39.1, 40.08, 44.96, 47.87, 50.94, 52.0, 54.94, 55.85, 58.93, 58.69, 63.55, 65.38, 69.72, 72.63, 74.92, 78.97, 79.9, 83.8, 85.47, 87.62, 88.91, 91.22, 92.91, 95.95, 98.0, 101.1, 102.9, 106.4, 107.9, 112.4, 114.8, 118.7, 121.8, 127.6, 126.9, 131.3, 132.9, 137.3, 138.9, 140.1, 140.9, 144.2, 145.0, 150.4, 152.0, 157.3, 158.9, 162.5, 164.9, 167.3, 168.9, 173.0, 175.0, 178.5, 180.9, 183.8, 186.2, 190.2, 192.2, 195.1, 197.0, 200.6, 204.4, 207.2, 209.0, 209.0, 210.0, 222.0, 223.0, 226.0, 227.0, 232.0, 231.0, 238.0, 237.0, 244.0, 243.0, 247.0, 247.0, 251.0, 252.0, 257.0, 258.0, 259.0, 262.0, 267.0, 270.0, 269.0, 270.0, 270.0, 278.0, 281.0, 281.0, 285.0, 286.0, 289.0, 289.0, 293.0, 293.0, 294.0]
_RADII = [0.0, 1.2, 1.4, 1.82, 1.53, 1.92, 1.7, 1.55, 1.52, 1.47, 1.54, 2.27, 1.73, 1.84, 2.1, 1.8, 1.8, 1.75, 1.88, 2.75, 2.31, 2.11, 2.0, 2.0, 2.0, 2.0, 2.0, 2.0, 1.63, 1.4, 1.39, 1.87, 2.11, 1.85, 1.9, 1.85, 2.02, 3.03, 2.49, 2.0, 2.0, 2.0, 2.0, 2.0, 2.0, 2.0, 1.63, 1.72, 1.58, 1.93, 2.17, 2.06, 2.06, 1.98, 2.16, 3.43, 2.68, 2.0, 2.0, 2.0, 2.0, 2.0, 2.0, 2.0, 2.0, 2.0, 2.0, 2.0, 2.0, 2.0, 2.0, 2.0, 2.0, 2.0, 2.0, 2.0, 2.0, 2.0, 1.75, 1.66, 1.55, 1.96, 2.02, 2.07, 1.97, 2.02, 2.2, 3.48, 2.83, 2.0, 2.0, 2.0, 1.86, 2.0, 2.0, 2.0, 2.0, 2.0, 2.0, 2.0, 2.0, 2.0, 2.0, 2.0, 2.0, 2.0, 2.0, 2.0, 2.0, 2.0, 2.0, 2.0, 2.0, 2.0, 2.0, 2.0, 2.0, 2.0, 2.0]
_EN = [0.0, 2.2, 0.0, 0.98, 1.57, 2.04, 2.55, 3.04, 3.44, 3.98, 0.0, 0.93, 1.31, 1.61, 1.9, 2.19, 2.58, 3.16, 0.0, 0.82, 1.0, 1.36, 1.54, 1.63, 1.66, 1.55, 1.83, 1.88, 1.91, 1.9, 1.65, 1.81, 2.01, 2.18, 2.55, 2.96, 3.0, 0.82, 0.95, 1.22, 1.33, 1.6, 2.16, 1.9, 2.2, 2.28, 2.2, 1.93, 1.69, 1.78, 1.96, 2.05, 2.1, 2.66, 2.6, 0.79, 0.89, 1.1, 1.12, 1.13, 1.14, 1.13, 1.17, 1.2, 1.2, 1.22, 1.23, 1.24, 1.25, 1.1, 1.27, 1.3, 1.5, 2.36, 1.9, 2.2, 2.2, 2.28, 2.54, 2.0, 1.62, 1.87, 2.33, 2.02, 2.0, 2.2, 2.2, 0.7, 0.9, 1.1, 1.3, 1.5, 1.38, 1.36, 1.28, 1.3, 1.3, 1.3, 1.3, 1.3, 1.3, 1.3, 1.3, 1.3, 1.3, 1.3, 1.3, 1.3, 1.3, 1.3, 1.3, 1.3, 1.3, 1.3, 1.3, 1.3, 1.3, 1.3, 1.3]
_IE = [0.0, 13.6, 24.59, 5.39, 9.32, 8.3, 11.26, 14.53, 13.62, 17.42, 21.56, 5.14, 7.65, 5.99, 8.15, 10.49, 10.36, 12.97, 15.76, 4.34, 6.11, 6.56, 6.83, 6.75, 6.77, 7.43, 7.9, 7.88, 7.64, 7.73, 9.39, 6.0, 7.9, 9.79, 9.75, 11.81, 14.0, 4.18, 5.69, 6.22, 6.63, 6.76, 7.09, 7.28, 7.36, 7.46, 8.34, 7.58, 8.99, 5.79, 7.34, 8.64, 9.01, 10.45, 12.13, 3.89, 5.21, 5.58, 5.54, 5.47, 5.53, 5.58, 5.64, 5.67, 6.15, 5.86, 5.94, 6.02, 6.11, 6.18, 6.25, 5.43, 6.83, 7.55, 7.86, 7.83, 8.44, 8.97, 8.96, 9.23, 10.44, 6.11, 7.42, 7.29, 8.42, 9.3, 10.75, 4.07, 5.28, 5.17, 6.31, 5.89, 6.19, 6.27, 6.03, 5.97, 6.02, 6.2, 6.28, 6.42, 6.5, 6.58, 6.65, 4.9, 6.0, 6.0, 6.0, 6.0, 6.0, 6.0, 6.0, 6.0, 6.0, 6.0, 6.0, 6.0, 6.0, 6.0, 6.0]

_PROPS = np.zeros((119, 4), np.float32)
_PROPS[: len(_MASSES), 0] = _MASSES
_PROPS[: len(_RADII), 1] = _RADII
_PROPS[: len(_EN), 2] = _EN
_PROPS[: len(_IE), 3] = _IE

N = 10000
E = 320000
D = 128
NB = 64
L = 4
B = 64
CUT = 8.0

NT = 10
TN = 1024
ET = 160
TE = E // ET

NWORK = 32
EW = E // NWORK
CH = 80
NCHUNK = EW // CH
NPAD = 10240
RPT = NPAD // 16


def _embed_body(an_ref, tab_ref, pw_ref, pb_ref, wm_ref, x_ref, p1_ref, p2_ref):
    ft = jnp.dot(tab_ref[...], pw_ref[...], preferred_element_type=jnp.float32)
    an = jnp.clip(an_ref[...], 0, 118)
    lanes = lax.broadcasted_iota(jnp.int32, (TN, 128), 1)
    oh = (an == lanes).astype(jnp.float32)
    x = jnp.dot(oh, ft, preferred_element_type=jnp.float32) + pb_ref[...]
    x_ref[...] = x
    p1_ref[...] = jnp.dot(x, wm_ref[:D], preferred_element_type=jnp.float32)
    p2_ref[...] = jnp.dot(x, wm_ref[D:], preferred_element_type=jnp.float32)


def _embed(an2d, tab_pad, proj_W, proj_b2d, wm12):
    return pl.pallas_call(
        _embed_body,
        grid=(NT,),
        in_specs=[
            pl.BlockSpec((TN, 1), lambda t: (t, 0)),
            pl.BlockSpec((128, 132), lambda t: (0, 0)),
            pl.BlockSpec((132, D), lambda t: (0, 0)),
            pl.BlockSpec((1, D), lambda t: (0, 0)),
            pl.BlockSpec((2 * D, D), lambda t: (0, 0)),
        ],
        out_specs=[
            pl.BlockSpec((TN, D), lambda t: (t, 0)),
            pl.BlockSpec((TN, D), lambda t: (t, 0)),
            pl.BlockSpec((TN, D), lambda t: (t, 0)),
        ],
        out_shape=[jax.ShapeDtypeStruct((NPAD, D), jnp.float32)] * 3,
    )(an2d, tab_pad, proj_W, proj_b2d, wm12)


def _eprep_body(d_ref, ew_ref, eb_ref, w3_ref, bm_ref, o0, o1, o2, o3):
    d = d_ref[...]
    mu = lax.broadcasted_iota(jnp.int32, (TE, NB), 1).astype(jnp.float32) * (
        CUT / (NB - 1)
    )
    sigma = CUT / NB
    z = d - mu
    rbf = jnp.exp(z * z * (-1.0 / (2.0 * sigma * sigma)))
    e = jnp.dot(rbf, ew_ref[...], preferred_element_type=jnp.float32) + eb_ref[...]
    e = jnp.maximum(e, 0.0)
    for l, o in enumerate((o0, o1, o2, o3)):
        o[...] = (
            jnp.dot(e, w3_ref[l], preferred_element_type=jnp.float32)
            + bm_ref[l, :, :]
        )


def _eprep(dist2d, edge_W, edge_b2d, w3_all, bm3d):
    return pl.pallas_call(
        _eprep_body,
        grid=(ET,),
        in_specs=[
            pl.BlockSpec((TE, 1), lambda t: (t, 0)),
            pl.BlockSpec((NB, D), lambda t: (0, 0)),
            pl.BlockSpec((1, D), lambda t: (0, 0)),
            pl.BlockSpec((L, D, D), lambda t: (0, 0, 0)),
            pl.BlockSpec((L, 1, D), lambda t: (0, 0, 0)),
        ],
        out_specs=[pl.BlockSpec((TE, D), lambda t: (t, 0))] * L,
        out_shape=[jax.ShapeDtypeStruct((E, D), jnp.float32)] * L,
    )(dist2d, edge_W, edge_b2d, w3_all, bm3d)


def _sc_body(p1_hbm, p2_hbm, e3_hbm, src_hbm, dst_hbm, zero_hbm,
             out0_hbm, out1_hbm,
             accum, sidx, didx, r1, r2, msg):
    c = lax.axis_index("c")
    s = lax.axis_index("s")
    wid = c * 16 + s
    base = wid * EW

    pltpu.sync_copy(zero_hbm.at[pl.ds(s * RPT, RPT)], accum.at[pl.ds(s * RPT, RPT)])
    plsc.subcore_barrier()

    def chunk(i, carry):
        b = base + i * CH
        pltpu.sync_copy(src_hbm.at[pl.ds(b, CH)], sidx)
        pltpu.sync_copy(dst_hbm.at[pl.ds(b, CH)], didx)
        pltpu.sync_copy(e3_hbm.at[pl.ds(b, CH)], msg)
        pltpu.sync_copy(p1_hbm.at[sidx], r1)
        pltpu.sync_copy(p2_hbm.at[didx], r2)

        def row(r, carry2):
            for k in range(D // 16):
                sl = pl.ds(k * 16, 16)
                v = r1[r, sl] + r2[r, sl] + msg[r, sl]
                msg[r, sl] = jnp.maximum(v, 0.0)
            return carry2

        lax.fori_loop(0, CH, row, 0)
        pltpu.sync_copy(msg, accum.at[didx], add=True)
        return carry

    lax.fori_loop(0, NCHUNK, chunk, 0)
    plsc.subcore_barrier()

    rows = pl.ds(s * RPT, RPT)

    @pl.when(c == 0)
    def _():
        pltpu.sync_copy(accum.at[rows], out0_hbm.at[rows])

    @pl.when(c == 1)
    def _():
        pltpu.sync_copy(accum.at[rows], out1_hbm.at[rows])


@functools.cache
def _sc_layer_fn():
    return pl.kernel(
        _sc_body,
        out_type=[jax.ShapeDtypeStruct((NPAD, D), jnp.float32)] * 2,
        mesh=plsc.VectorSubcoreMesh(
            core_axis_name="c", subcore_axis_name="s", num_cores=2,
            num_subcores=16,
        ),
        scratch_types=[
            pltpu.VMEM_SHARED((NPAD, D), jnp.float32),
            pltpu.VMEM((CH,), jnp.int32),
            pltpu.VMEM((CH,), jnp.int32),
            pltpu.VMEM((CH, D), jnp.float32),
            pltpu.VMEM((CH, D), jnp.float32),
            pltpu.VMEM((CH, D), jnp.float32),
        ],
    )


def _sc_layer(p1, p2, e3l, src, dst, zero_nd):
    return _sc_layer_fn()(p1, p2, e3l, src, dst, zero_nd)


def _upd_body(x_ref, a0_ref, a1_ref, wu_ref, bu_ref, wm_ref, xn_ref, p1_ref, p2_ref):
    x = x_ref[...]
    a = a0_ref[...] + a1_ref[...]
    u = (
        jnp.dot(x, wu_ref[:D], preferred_element_type=jnp.float32)
        + jnp.dot(a, wu_ref[D:], preferred_element_type=jnp.float32)
        + bu_ref[...]
    )
    xn = x + jnp.maximum(u, 0.0)
    xn_ref[...] = xn
    p1_ref[...] = jnp.dot(xn, wm_ref[:D], preferred_element_type=jnp.float32)
    p2_ref[...] = jnp.dot(xn, wm_ref[D:], preferred_element_type=jnp.float32)


def _update(x, a0, a1, wu, bu2d, wm12):
    return pl.pallas_call(
        _upd_body,
        grid=(NT,),
        in_specs=[
            pl.BlockSpec((TN, D), lambda t: (t, 0)),
            pl.BlockSpec((TN, D), lambda t: (t, 0)),
            pl.BlockSpec((TN, D), lambda t: (t, 0)),
            pl.BlockSpec((2 * D, D), lambda t: (0, 0)),
            pl.BlockSpec((1, D), lambda t: (0, 0)),
            pl.BlockSpec((2 * D, D), lambda t: (0, 0)),
        ],
        out_specs=[pl.BlockSpec((TN, D), lambda t: (t, 0))] * 3,
        out_shape=[jax.ShapeDtypeStruct((NPAD, D), jnp.float32)] * 3,
    )(x, a0, a1, wu, bu2d, wm12)


def _read_body(x_ref, b_ref, w1_ref, b1_ref, w2_ref, b2_ref, o_ref, pool_ref):
    t = pl.program_id(0)
    seg = lax.broadcasted_iota(jnp.int32, (TN, B), 1)
    oh = (b_ref[...] == seg).astype(jnp.float32)
    part = lax.dot_general(
        oh, x_ref[...], (((0,), (0,)), ((), ())),
        preferred_element_type=jnp.float32,
    )

    @pl.when(t == 0)
    def _():
        pool_ref[...] = part

    @pl.when(t > 0)
    def _():
        pool_ref[...] = pool_ref[...] + part

    @pl.when(t == NT - 1)
    def _():
        h = jnp.dot(pool_ref[...], w1_ref[...], preferred_element_type=jnp.float32)
        h = jnp.maximum(h + b1_ref[...], 0.0)
        o = jnp.sum(h * w2_ref[...], axis=1, keepdims=True) + b2_ref[...]
        o_ref[...] = o


def _readout(x, batch2d, out_W1, out_b12d, w2row, out_b22d):
    return pl.pallas_call(
        _read_body,
        grid=(NT,),
        in_specs=[
            pl.BlockSpec((TN, D), lambda t: (t, 0)),
            pl.BlockSpec((TN, 1), lambda t: (t, 0)),
            pl.BlockSpec((D, D), lambda t: (0, 0)),
            pl.BlockSpec((1, D), lambda t: (0, 0)),
            pl.BlockSpec((1, D), lambda t: (0, 0)),
            pl.BlockSpec((1, 1), lambda t: (0, 0)),
        ],
        out_specs=pl.BlockSpec((B, 1), lambda t: (0, 0)),
        out_shape=jax.ShapeDtypeStruct((B, 1), jnp.float32),
        scratch_shapes=[pltpu.VMEM((B, D), jnp.float32)],
    )(x, batch2d, out_W1, out_b12d, w2row, out_b22d)


def kernel(atomic_numbers, edge_index, edge_dist, batch, emb_table, proj_W,
           proj_b, edge_W, edge_b, Wm, bm, Wu, bu, out_W1, out_b1, out_W2,
           out_b2):
    f32 = jnp.float32
    an2d = jnp.zeros((NPAD, 1), jnp.int32).at[:N].set(
        atomic_numbers.astype(jnp.int32).reshape(N, 1))
    src = edge_index[0].astype(jnp.int32)
    dst = edge_index[1].astype(jnp.int32)
    batch2d = jnp.full((NPAD, 1), B, jnp.int32).at[:N].set(
        batch.astype(jnp.int32).reshape(N, 1))

    tab = jnp.concatenate([emb_table, jnp.asarray(_PROPS)], axis=1)
    tab_pad = jnp.zeros((128, 132), f32).at[:119].set(tab)
    proj_b2d = proj_b.reshape(1, D)
    edge_b2d = edge_b.reshape(1, D)
    dist2d = edge_dist.reshape(E, 1)
    zero_nd = jnp.zeros((NPAD, D), f32)

    x, p1, p2 = _embed(an2d, tab_pad, proj_W, proj_b2d, Wm[0, : 2 * D])
    e3 = _eprep(dist2d, edge_W, edge_b2d, Wm[:, 2 * D :, :], bm.reshape(L, 1, D))

    for l in range(L):
        a0, a1 = _sc_layer(p1, p2, e3[l], src, dst, zero_nd)
        wm_next = Wm[(l + 1) % L, : 2 * D]
        x, p1, p2 = _update(x, a0, a1, Wu[l], bu[l].reshape(1, D), wm_next)

    return _readout(x, batch2d, out_W1, out_b1.reshape(1, D),
                    out_W2.reshape(1, D), out_b2.reshape(1, 1))


# speedup vs baseline: 6.0147x; 1.9020x over previous
import functools

import jax
import jax.numpy as jnp
import numpy as np
from jax import lax
from jax.experimental import pallas as pl
from jax.experimental.pallas import tpu as pltpu
from jax.experimental.pallas import tpu_sc as plsc

_MASSES = [0.0, 1.008, 4.003, 6.941, 9.012, 10.81, 12.01, 14.01, 16.0, 19.0, 20.18, 22.99, 24.31, 26.98, 28.09, 30.97, 32.07, 35.45, 39.95, 39.1, 40.08, 44.96, 47.87, 50.94, 52.0, 54.94, 55.85, 58.93, 58.69, 63.55, 65.38, 69.72, 72.63, 74.92, 78.97, 79.9, 83.8, 85.47, 87.62, 88.91, 91.22, 92.91, 95.95, 98.0, 101.1, 102.9, 106.4, 107.9, 112.4, 114.8, 118.7, 121.8, 127.6, 126.9, 131.3, 132.9, 137.3, 138.9, 140.1, 140.9, 144.2, 145.0, 150.4, 152.0, 157.3, 158.9, 162.5, 164.9, 167.3, 168.9, 173.0, 175.0, 178.5, 180.9, 183.8, 186.2, 190.2, 192.2, 195.1, 197.0, 200.6, 204.4, 207.2, 209.0, 209.0, 210.0, 222.0, 223.0, 226.0, 227.0, 232.0, 231.0, 238.0, 237.0, 244.0, 243.0, 247.0, 247.0, 251.0, 252.0, 257.0, 258.0, 259.0, 262.0, 267.0, 270.0, 269.0, 270.0, 270.0, 278.0, 281.0, 281.0, 285.0, 286.0, 289.0, 289.0, 293.0, 293.0, 294.0]
_RADII = [0.0, 1.2, 1.4, 1.82, 1.53, 1.92, 1.7, 1.55, 1.52, 1.47, 1.54, 2.27, 1.73, 1.84, 2.1, 1.8, 1.8, 1.75, 1.88, 2.75, 2.31, 2.11, 2.0, 2.0, 2.0, 2.0, 2.0, 2.0, 1.63, 1.4, 1.39, 1.87, 2.11, 1.85, 1.9, 1.85, 2.02, 3.03, 2.49, 2.0, 2.0, 2.0, 2.0, 2.0, 2.0, 2.0, 1.63, 1.72, 1.58, 1.93, 2.17, 2.06, 2.06, 1.98, 2.16, 3.43, 2.68, 2.0, 2.0, 2.0, 2.0, 2.0, 2.0, 2.0, 2.0, 2.0, 2.0, 2.0, 2.0, 2.0, 2.0, 2.0, 2.0, 2.0, 2.0, 2.0, 2.0, 2.0, 1.75, 1.66, 1.55, 1.96, 2.02, 2.07, 1.97, 2.02, 2.2, 3.48, 2.83, 2.0, 2.0, 2.0, 1.86, 2.0, 2.0, 2.0, 2.0, 2.0, 2.0, 2.0, 2.0, 2.0, 2.0, 2.0, 2.0, 2.0, 2.0, 2.0, 2.0, 2.0, 2.0, 2.0, 2.0, 2.0, 2.0, 2.0, 2.0, 2.0, 2.0]
_EN = [0.0, 2.2, 0.0, 0.98, 1.57, 2.04, 2.55, 3.04, 3.44, 3.98, 0.0, 0.93, 1.31, 1.61, 1.9, 2.19, 2.58, 3.16, 0.0, 0.82, 1.0, 1.36, 1.54, 1.63, 1.66, 1.55, 1.83, 1.88, 1.91, 1.9, 1.65, 1.81, 2.01, 2.18, 2.55, 2.96, 3.0, 0.82, 0.95, 1.22, 1.33, 1.6, 2.16, 1.9, 2.2, 2.28, 2.2, 1.93, 1.69, 1.78, 1.96, 2.05, 2.1, 2.66, 2.6, 0.79, 0.89, 1.1, 1.12, 1.13, 1.14, 1.13, 1.17, 1.2, 1.2, 1.22, 1.23, 1.24, 1.25, 1.1, 1.27, 1.3, 1.5, 2.36, 1.9, 2.2, 2.2, 2.28, 2.54, 2.0, 1.62, 1.87, 2.33, 2.02, 2.0, 2.2, 2.2, 0.7, 0.9, 1.1, 1.3, 1.5, 1.38, 1.36, 1.28, 1.3, 1.3, 1.3, 1.3, 1.3, 1.3, 1.3, 1.3, 1.3, 1.3, 1.3, 1.3, 1.3, 1.3, 1.3, 1.3, 1.3, 1.3, 1.3, 1.3, 1.3, 1.3, 1.3, 1.3]
_IE = [0.0, 13.6, 24.59, 5.39, 9.32, 8.3, 11.26, 14.53, 13.62, 17.42, 21.56, 5.14, 7.65, 5.99, 8.15, 10.49, 10.36, 12.97, 15.76, 4.34, 6.11, 6.56, 6.83, 6.75, 6.77, 7.43, 7.9, 7.88, 7.64, 7.73, 9.39, 6.0, 7.9, 9.79, 9.75, 11.81, 14.0, 4.18, 5.69, 6.22, 6.63, 6.76, 7.09, 7.28, 7.36, 7.46, 8.34, 7.58, 8.99, 5.79, 7.34, 8.64, 9.01, 10.45, 12.13, 3.89, 5.21, 5.58, 5.54, 5.47, 5.53, 5.58, 5.64, 5.67, 6.15, 5.86, 5.94, 6.02, 6.11, 6.18, 6.25, 5.43, 6.83, 7.55, 7.86, 7.83, 8.44, 8.97, 8.96, 9.23, 10.44, 6.11, 7.42, 7.29, 8.42, 9.3, 10.75, 4.07, 5.28, 5.17, 6.31, 5.89, 6.19, 6.27, 6.03, 5.97, 6.02, 6.2, 6.28, 6.42, 6.5, 6.58, 6.65, 4.9, 6.0, 6.0, 6.0, 6.0, 6.0, 6.0, 6.0, 6.0, 6.0, 6.0, 6.0, 6.0, 6.0, 6.0, 6.0]

_PROPS = np.zeros((119, 4), np.float32)
_PROPS[: len(_MASSES), 0] = _MASSES
_PROPS[: len(_RADII), 1] = _RADII
_PROPS[: len(_EN), 2] = _EN
_PROPS[: len(_IE), 3] = _IE

N = 10000
E = 320000
D = 128
NB = 64
L = 4
B = 64
CUT = 8.0

NT = 10
TN = 1024
ET = 160
TE = E // ET

NWORK = 32
EW = E // NWORK
CH = 40
NCHUNK = EW // CH
NPAD = 10240
RPT = NPAD // 16
NH = 50


def _embed_body(an_ref, tab_ref, pw_ref, pb_ref, wm_ref, x_ref, p1_ref, p2_ref):
    ft = jnp.dot(tab_ref[...], pw_ref[...], preferred_element_type=jnp.float32)
    an = jnp.clip(an_ref[...], 0, 118)
    lanes = lax.broadcasted_iota(jnp.int32, (TN, 128), 1)
    oh = (an == lanes).astype(jnp.float32)
    x = jnp.dot(oh, ft, preferred_element_type=jnp.float32) + pb_ref[...]
    x_ref[...] = x
    p1_ref[...] = jnp.dot(x, wm_ref[:D], preferred_element_type=jnp.float32)
    p2_ref[...] = jnp.dot(x, wm_ref[D:], preferred_element_type=jnp.float32)


def _embed(an2d, tab_pad, proj_W, proj_b2d, wm12):
    return pl.pallas_call(
        _embed_body,
        grid=(NT,),
        in_specs=[
            pl.BlockSpec((TN, 1), lambda t: (t, 0)),
            pl.BlockSpec((128, 132), lambda t: (0, 0)),
            pl.BlockSpec((132, D), lambda t: (0, 0)),
            pl.BlockSpec((1, D), lambda t: (0, 0)),
            pl.BlockSpec((2 * D, D), lambda t: (0, 0)),
        ],
        out_specs=[
            pl.BlockSpec((TN, D), lambda t: (t, 0)),
            pl.BlockSpec((TN, D), lambda t: (t, 0)),
            pl.BlockSpec((TN, D), lambda t: (t, 0)),
        ],
        out_shape=[jax.ShapeDtypeStruct((NPAD, D), jnp.float32)] * 3,
    )(an2d, tab_pad, proj_W, proj_b2d, wm12)


def _eprep_body(d_ref, ew_ref, eb_ref, w3_ref, bm_ref, o0, o1, o2, o3):
    d = d_ref[...]
    mu = lax.broadcasted_iota(jnp.int32, (TE, NB), 1).astype(jnp.float32) * (
        CUT / (NB - 1)
    )
    sigma = CUT / NB
    z = d - mu
    rbf = jnp.exp(z * z * (-1.0 / (2.0 * sigma * sigma)))
    e = jnp.dot(rbf, ew_ref[...], preferred_element_type=jnp.float32) + eb_ref[...]
    e = jnp.maximum(e, 0.0)
    for l, o in enumerate((o0, o1, o2, o3)):
        o[...] = (
            jnp.dot(e, w3_ref[l], preferred_element_type=jnp.float32)
            + bm_ref[l, :, :]
        )


def _eprep(dist2d, edge_W, edge_b2d, w3_all, bm3d):
    return pl.pallas_call(
        _eprep_body,
        grid=(ET,),
        in_specs=[
            pl.BlockSpec((TE, 1), lambda t: (t, 0)),
            pl.BlockSpec((NB, D), lambda t: (0, 0)),
            pl.BlockSpec((1, D), lambda t: (0, 0)),
            pl.BlockSpec((L, D, D), lambda t: (0, 0, 0)),
            pl.BlockSpec((L, 1, D), lambda t: (0, 0, 0)),
        ],
        out_specs=[pl.BlockSpec((TE, D), lambda t: (t, 0))] * L,
        out_shape=[jax.ShapeDtypeStruct((E, D), jnp.float32)] * L,
    )(dist2d, edge_W, edge_b2d, w3_all, bm3d)


def _sc_body(p1_hbm, p2_hbm, e3_hbm, src_hbm, dst3_hbm, zero_hbm,
             out0_hbm, out1_hbm,
             accum, srcv, dstv,
             r1a, r2a, msga, r1b, r2b, msgb,
             sga, sgb, ssa, ssb):
    c = lax.axis_index("c")
    s = lax.axis_index("s")
    wid = c * 16 + s
    ebase = wid * EW

    pltpu.sync_copy(zero_hbm.at[pl.ds(s * RPT, RPT)], accum.at[pl.ds(s * RPT, RPT)])
    pltpu.sync_copy(src_hbm.at[pl.ds(ebase, NH * CH)], srcv)
    pltpu.sync_copy(dst3_hbm.at[wid, 0], dstv)
    plsc.subcore_barrier()

    def issue(i, r1, r2, msg, sem):
        li = lax.rem(i, NH)
        pltpu.async_copy(e3_hbm.at[pl.ds(ebase + i * CH, CH)], msg, sem)
        pltpu.async_copy(p1_hbm.at[srcv.at[pl.ds(li * CH, CH)]], r1, sem)
        pltpu.async_copy(p2_hbm.at[dstv.at[li]], r2, sem)

    def drain(sem, n):
        for _ in range(n):
            pltpu.make_async_copy(e3_hbm.at[pl.ds(0, CH)], msga, sem).wait()

    def compute(r1, r2, msg):
        @plsc.parallel_loop(0, CH, 1, unroll=2)
        def _row(r):
            for k in range(D // 16):
                sl = pl.ds(k * 16, 16)
                msg[r, sl] = jnp.maximum(r1[r, sl] + r2[r, sl] + msg[r, sl], 0.0)

    issue(0, r1a, r2a, msga, sga)

    def body(i, carry):
        mi = lax.rem(i, NH)
        refill = jnp.logical_and(mi == NH - 1, i + 1 < NCHUNK)

        @pl.when(jnp.logical_not(refill))
        def _():
            @pl.when(i + 1 < NCHUNK)
            def _():
                @pl.when(i % 2 == 0)
                def _():
                    issue(i + 1, r1b, r2b, msgb, sgb)

                @pl.when(i % 2 == 1)
                def _():
                    issue(i + 1, r1a, r2a, msga, sga)

            @pl.when(i % 2 == 0)
            def _():
                drain(sga, 3)
                compute(r1a, r2a, msga)
                pltpu.async_copy(msga, accum.at[dstv.at[mi]], ssa, add=True)
                drain(ssa, 1)

            @pl.when(i % 2 == 1)
            def _():
                drain(sgb, 3)
                compute(r1b, r2b, msgb)
                pltpu.async_copy(msgb, accum.at[dstv.at[mi]], ssb, add=True)
                drain(ssb, 1)

        @pl.when(refill)
        def _():
            drain(sgb, 3)
            compute(r1b, r2b, msgb)
            pltpu.async_copy(msgb, accum.at[dstv.at[mi]], ssb, add=True)
            drain(ssb, 1)
            h = (i + 1) // NH
            pltpu.sync_copy(src_hbm.at[pl.ds(ebase + (i + 1) * CH, NH * CH)], srcv)
            pltpu.sync_copy(dst3_hbm.at[wid, h], dstv)
            issue(i + 1, r1a, r2a, msga, sga)

        return carry

    lax.fori_loop(0, NCHUNK, body, 0)
    plsc.subcore_barrier()

    rows = pl.ds(s * RPT, RPT)

    @pl.when(c == 0)
    def _():
        pltpu.sync_copy(accum.at[rows], out0_hbm.at[rows])

    @pl.when(c == 1)
    def _():
        pltpu.sync_copy(accum.at[rows], out1_hbm.at[rows])


@functools.cache
def _sc_layer_fn():
    return pl.kernel(
        _sc_body,
        out_type=[jax.ShapeDtypeStruct((NPAD, D), jnp.float32)] * 2,
        mesh=plsc.VectorSubcoreMesh(
            core_axis_name="c", subcore_axis_name="s", num_cores=2,
            num_subcores=16,
        ),
        scratch_types=[
            pltpu.VMEM_SHARED((NPAD, D), jnp.float32),
            pltpu.VMEM((NH * CH,), jnp.int32),
            pltpu.VMEM((NH, CH), jnp.int32),
            pltpu.VMEM((CH, D), jnp.float32),
            pltpu.VMEM((CH, D), jnp.float32),
            pltpu.VMEM((CH, D), jnp.float32),
            pltpu.VMEM((CH, D), jnp.float32),
            pltpu.VMEM((CH, D), jnp.float32),
            pltpu.VMEM((CH, D), jnp.float32),
            pltpu.SemaphoreType.DMA,
            pltpu.SemaphoreType.DMA,
            pltpu.SemaphoreType.DMA,
            pltpu.SemaphoreType.DMA,
        ],
    )


def _sc_layer(p1, p2, e3l, src, dst3, zero_nd):
    return _sc_layer_fn()(p1, p2, e3l, src, dst3, zero_nd)


def _upd_body(x_ref, a0_ref, a1_ref, wu_ref, bu_ref, wm_ref, xn_ref, p1_ref, p2_ref):
    x = x_ref[...]
    a = a0_ref[...] + a1_ref[...]
    u = (
        jnp.dot(x, wu_ref[:D], preferred_element_type=jnp.float32)
        + jnp.dot(a, wu_ref[D:], preferred_element_type=jnp.float32)
        + bu_ref[...]
    )
    xn = x + jnp.maximum(u, 0.0)
    xn_ref[...] = xn
    p1_ref[...] = jnp.dot(xn, wm_ref[:D], preferred_element_type=jnp.float32)
    p2_ref[...] = jnp.dot(xn, wm_ref[D:], preferred_element_type=jnp.float32)


def _update(x, a0, a1, wu, bu2d, wm12):
    return pl.pallas_call(
        _upd_body,
        grid=(NT,),
        in_specs=[
            pl.BlockSpec((TN, D), lambda t: (t, 0)),
            pl.BlockSpec((TN, D), lambda t: (t, 0)),
            pl.BlockSpec((TN, D), lambda t: (t, 0)),
            pl.BlockSpec((2 * D, D), lambda t: (0, 0)),
            pl.BlockSpec((1, D), lambda t: (0, 0)),
            pl.BlockSpec((2 * D, D), lambda t: (0, 0)),
        ],
        out_specs=[pl.BlockSpec((TN, D), lambda t: (t, 0))] * 3,
        out_shape=[jax.ShapeDtypeStruct((NPAD, D), jnp.float32)] * 3,
    )(x, a0, a1, wu, bu2d, wm12)


def _read_body(x_ref, b_ref, w1_ref, b1_ref, w2_ref, b2_ref, o_ref, pool_ref):
    t = pl.program_id(0)
    seg = lax.broadcasted_iota(jnp.int32, (TN, B), 1)
    oh = (b_ref[...] == seg).astype(jnp.float32)
    part = lax.dot_general(
        oh, x_ref[...], (((0,), (0,)), ((), ())),
        preferred_element_type=jnp.float32,
    )

    @pl.when(t == 0)
    def _():
        pool_ref[...] = part

    @pl.when(t > 0)
    def _():
        pool_ref[...] = pool_ref[...] + part

    @pl.when(t == NT - 1)
    def _():
        h = jnp.dot(pool_ref[...], w1_ref[...], preferred_element_type=jnp.float32)
        h = jnp.maximum(h + b1_ref[...], 0.0)
        o = jnp.sum(h * w2_ref[...], axis=1, keepdims=True) + b2_ref[...]
        o_ref[...] = o


def _readout(x, batch2d, out_W1, out_b12d, w2row, out_b22d):
    return pl.pallas_call(
        _read_body,
        grid=(NT,),
        in_specs=[
            pl.BlockSpec((TN, D), lambda t: (t, 0)),
            pl.BlockSpec((TN, 1), lambda t: (t, 0)),
            pl.BlockSpec((D, D), lambda t: (0, 0)),
            pl.BlockSpec((1, D), lambda t: (0, 0)),
            pl.BlockSpec((1, D), lambda t: (0, 0)),
            pl.BlockSpec((1, 1), lambda t: (0, 0)),
        ],
        out_specs=pl.BlockSpec((B, 1), lambda t: (0, 0)),
        out_shape=jax.ShapeDtypeStruct((B, 1), jnp.float32),
        scratch_shapes=[pltpu.VMEM((B, D), jnp.float32)],
    )(x, batch2d, out_W1, out_b12d, w2row, out_b22d)


def kernel(atomic_numbers, edge_index, edge_dist, batch, emb_table, proj_W,
           proj_b, edge_W, edge_b, Wm, bm, Wu, bu, out_W1, out_b1, out_W2,
           out_b2):
    f32 = jnp.float32
    an2d = jnp.zeros((NPAD, 1), jnp.int32).at[:N].set(
        atomic_numbers.astype(jnp.int32).reshape(N, 1))
    src = edge_index[0].astype(jnp.int32)
    dst3 = edge_index[1].astype(jnp.int32).reshape(NWORK, NCHUNK // NH, NH, CH)
    batch2d = jnp.full((NPAD, 1), B, jnp.int32).at[:N].set(
        batch.astype(jnp.int32).reshape(N, 1))

    tab = jnp.concatenate([emb_table, jnp.asarray(_PROPS)], axis=1)
    tab_pad = jnp.zeros((128, 132), f32).at[:119].set(tab)
    proj_b2d = proj_b.reshape(1, D)
    edge_b2d = edge_b.reshape(1, D)
    dist2d = edge_dist.reshape(E, 1)
    zero_nd = jnp.zeros((NPAD, D), f32)

    x, p1, p2 = _embed(an2d, tab_pad, proj_W, proj_b2d, Wm[0, : 2 * D])
    e3 = _eprep(dist2d, edge_W, edge_b2d, Wm[:, 2 * D :, :], bm.reshape(L, 1, D))

    for l in range(L):
        a0, a1 = _sc_layer(p1, p2, e3[l], src, dst3, zero_nd)
        wm_next = Wm[(l + 1) % L, : 2 * D]
        x, p1, p2 = _update(x, a0, a1, Wu[l], bu[l].reshape(1, D), wm_next)

    return _readout(x, batch2d, out_W1, out_b1.reshape(1, D),
                    out_W2.reshape(1, D), out_b2.reshape(1, 1))


# speedup vs baseline: 6.0581x; 1.0072x over previous
import functools

import jax
import jax.numpy as jnp
import numpy as np
from jax import lax
from jax.experimental import pallas as pl
from jax.experimental.pallas import tpu as pltpu
from jax.experimental.pallas import tpu_sc as plsc

_MASSES = [0.0, 1.008, 4.003, 6.941, 9.012, 10.81, 12.01, 14.01, 16.0, 19.0, 20.18, 22.99, 24.31, 26.98, 28.09, 30.97, 32.07, 35.45, 39.95, 39.1, 40.08, 44.96, 47.87, 50.94, 52.0, 54.94, 55.85, 58.93, 58.69, 63.55, 65.38, 69.72, 72.63, 74.92, 78.97, 79.9, 83.8, 85.47, 87.62, 88.91, 91.22, 92.91, 95.95, 98.0, 101.1, 102.9, 106.4, 107.9, 112.4, 114.8, 118.7, 121.8, 127.6, 126.9, 131.3, 132.9, 137.3, 138.9, 140.1, 140.9, 144.2, 145.0, 150.4, 152.0, 157.3, 158.9, 162.5, 164.9, 167.3, 168.9, 173.0, 175.0, 178.5, 180.9, 183.8, 186.2, 190.2, 192.2, 195.1, 197.0, 200.6, 204.4, 207.2, 209.0, 209.0, 210.0, 222.0, 223.0, 226.0, 227.0, 232.0, 231.0, 238.0, 237.0, 244.0, 243.0, 247.0, 247.0, 251.0, 252.0, 257.0, 258.0, 259.0, 262.0, 267.0, 270.0, 269.0, 270.0, 270.0, 278.0, 281.0, 281.0, 285.0, 286.0, 289.0, 289.0, 293.0, 293.0, 294.0]
_RADII = [0.0, 1.2, 1.4, 1.82, 1.53, 1.92, 1.7, 1.55, 1.52, 1.47, 1.54, 2.27, 1.73, 1.84, 2.1, 1.8, 1.8, 1.75, 1.88, 2.75, 2.31, 2.11, 2.0, 2.0, 2.0, 2.0, 2.0, 2.0, 1.63, 1.4, 1.39, 1.87, 2.11, 1.85, 1.9, 1.85, 2.02, 3.03, 2.49, 2.0, 2.0, 2.0, 2.0, 2.0, 2.0, 2.0, 1.63, 1.72, 1.58, 1.93, 2.17, 2.06, 2.06, 1.98, 2.16, 3.43, 2.68, 2.0, 2.0, 2.0, 2.0, 2.0, 2.0, 2.0, 2.0, 2.0, 2.0, 2.0, 2.0, 2.0, 2.0, 2.0, 2.0, 2.0, 2.0, 2.0, 2.0, 2.0, 1.75, 1.66, 1.55, 1.96, 2.02, 2.07, 1.97, 2.02, 2.2, 3.48, 2.83, 2.0, 2.0, 2.0, 1.86, 2.0, 2.0, 2.0, 2.0, 2.0, 2.0, 2.0, 2.0, 2.0, 2.0, 2.0, 2.0, 2.0, 2.0, 2.0, 2.0, 2.0, 2.0, 2.0, 2.0, 2.0, 2.0, 2.0, 2.0, 2.0, 2.0]
_EN = [0.0, 2.2, 0.0, 0.98, 1.57, 2.04, 2.55, 3.04, 3.44, 3.98, 0.0, 0.93, 1.31, 1.61, 1.9, 2.19, 2.58, 3.16, 0.0, 0.82, 1.0, 1.36, 1.54, 1.63, 1.66, 1.55, 1.83, 1.88, 1.91, 1.9, 1.65, 1.81, 2.01, 2.18, 2.55, 2.96, 3.0, 0.82, 0.95, 1.22, 1.33, 1.6, 2.16, 1.9, 2.2, 2.28, 2.2, 1.93, 1.69, 1.78, 1.96, 2.05, 2.1, 2.66, 2.6, 0.79, 0.89, 1.1, 1.12, 1.13, 1.14, 1.13, 1.17, 1.2, 1.2, 1.22, 1.23, 1.24, 1.25, 1.1, 1.27, 1.3, 1.5, 2.36, 1.9, 2.2, 2.2, 2.28, 2.54, 2.0, 1.62, 1.87, 2.33, 2.02, 2.0, 2.2, 2.2, 0.7, 0.9, 1.1, 1.3, 1.5, 1.38, 1.36, 1.28, 1.3, 1.3, 1.3, 1.3, 1.3, 1.3, 1.3, 1.3, 1.3, 1.3, 1.3, 1.3, 1.3, 1.3, 1.3, 1.3, 1.3, 1.3, 1.3, 1.3, 1.3, 1.3, 1.3, 1.3]
_IE = [0.0, 13.6, 24.59, 5.39, 9.32, 8.3, 11.26, 14.53, 13.62, 17.42, 21.56, 5.14, 7.65, 5.99, 8.15, 10.49, 10.36, 12.97, 15.76, 4.34, 6.11, 6.56, 6.83, 6.75, 6.77, 7.43, 7.9, 7.88, 7.64, 7.73, 9.39, 6.0, 7.9, 9.79, 9.75, 11.81, 14.0, 4.18, 5.69, 6.22, 6.63, 6.76, 7.09, 7.28, 7.36, 7.46, 8.34, 7.58, 8.99, 5.79, 7.34, 8.64, 9.01, 10.45, 12.13, 3.89, 5.21, 5.58, 5.54, 5.47, 5.53, 5.58, 5.64, 5.67, 6.15, 5.86, 5.94, 6.02, 6.11, 6.18, 6.25, 5.43, 6.83, 7.55, 7.86, 7.83, 8.44, 8.97, 8.96, 9.23, 10.44, 6.11, 7.42, 7.29, 8.42, 9.3, 10.75, 4.07, 5.28, 5.17, 6.31, 5.89, 6.19, 6.27, 6.03, 5.97, 6.02, 6.2, 6.28, 6.42, 6.5, 6.58, 6.65, 4.9, 6.0, 6.0, 6.0, 6.0, 6.0, 6.0, 6.0, 6.0, 6.0, 6.0, 6.0, 6.0, 6.0, 6.0, 6.0]

_PROPS = np.zeros((119, 4), np.float32)
_PROPS[: len(_MASSES), 0] = _MASSES
_PROPS[: len(_RADII), 1] = _RADII
_PROPS[: len(_EN), 2] = _EN
_PROPS[: len(_IE), 3] = _IE

N = 10000
E = 320000
D = 128
NB = 64
L = 4
B = 64
CUT = 8.0

NT = 10
TN = 1024
ET = 160
TE = E // ET

NWORK = 32
EW = E // NWORK
CH = 40
NCHUNK = EW // CH
NPAD = 10240
RPT = NPAD // 16
NH = 50


def _embed_body(an_ref, tab_ref, pw_ref, pb_ref, wm_ref, x_ref, p1_ref, p2_ref):
    ft = jnp.dot(tab_ref[...], pw_ref[...], preferred_element_type=jnp.float32)
    an = jnp.clip(an_ref[...], 0, 118)
    lanes = lax.broadcasted_iota(jnp.int32, (TN, 128), 1)
    oh = (an == lanes).astype(jnp.float32)
    x = jnp.dot(oh, ft, preferred_element_type=jnp.float32) + pb_ref[...]
    x_ref[...] = x
    p1_ref[...] = jnp.dot(x, wm_ref[:D], preferred_element_type=jnp.float32)
    p2_ref[...] = jnp.dot(x, wm_ref[D:], preferred_element_type=jnp.float32)


def _embed(an2d, tab_pad, proj_W, proj_b2d, wm12):
    return pl.pallas_call(
        _embed_body,
        grid=(NT,),
        in_specs=[
            pl.BlockSpec((TN, 1), lambda t: (t, 0)),
            pl.BlockSpec((128, 132), lambda t: (0, 0)),
            pl.BlockSpec((132, D), lambda t: (0, 0)),
            pl.BlockSpec((1, D), lambda t: (0, 0)),
            pl.BlockSpec((2 * D, D), lambda t: (0, 0)),
        ],
        out_specs=[
            pl.BlockSpec((TN, D), lambda t: (t, 0)),
            pl.BlockSpec((TN, D), lambda t: (t, 0)),
            pl.BlockSpec((TN, D), lambda t: (t, 0)),
        ],
        out_shape=[jax.ShapeDtypeStruct((NPAD, D), jnp.float32)] * 3,
    )(an2d, tab_pad, proj_W, proj_b2d, wm12)


def _eprep_body(d_ref, ew_ref, eb_ref, w3_ref, bm_ref, o_ref):
    d = d_ref[...]
    mu = lax.broadcasted_iota(jnp.int32, (TE, NB), 1).astype(jnp.float32) * (
        CUT / (NB - 1)
    )
    sigma = CUT / NB
    z = d - mu
    rbf = jnp.exp(z * z * (-1.0 / (2.0 * sigma * sigma)))
    e = jnp.dot(rbf, ew_ref[...], preferred_element_type=jnp.float32) + eb_ref[...]
    e = jnp.maximum(e, 0.0)
    o_ref[...] = (
        jnp.dot(e, w3_ref[...], preferred_element_type=jnp.float32) + bm_ref[...]
    )


def _eprep(dist2d, edge_W, edge_b2d, w3, bm2d):
    return pl.pallas_call(
        _eprep_body,
        grid=(ET,),
        in_specs=[
            pl.BlockSpec((TE, 1), lambda t: (t, 0)),
            pl.BlockSpec((NB, D), lambda t: (0, 0)),
            pl.BlockSpec((1, D), lambda t: (0, 0)),
            pl.BlockSpec((D, D), lambda t: (0, 0)),
            pl.BlockSpec((1, D), lambda t: (0, 0)),
        ],
        out_specs=pl.BlockSpec((TE, D), lambda t: (t, 0)),
        out_shape=jax.ShapeDtypeStruct((E, D), jnp.float32),
    )(dist2d, edge_W, edge_b2d, w3, bm2d)


def _sc_body(p1_hbm, p2_hbm, e3_hbm, src_hbm, dst3_hbm, zero_hbm,
             out0_hbm, out1_hbm,
             accum, srcv, dstv,
             r1a, r2a, msga, r1b, r2b, msgb,
             sga, sgb, ssa, ssb):
    c = lax.axis_index("c")
    s = lax.axis_index("s")
    wid = c * 16 + s
    ebase = wid * EW

    pltpu.sync_copy(zero_hbm.at[pl.ds(s * RPT, RPT)], accum.at[pl.ds(s * RPT, RPT)])
    pltpu.sync_copy(src_hbm.at[pl.ds(ebase, NH * CH)], srcv)
    pltpu.sync_copy(dst3_hbm.at[wid, 0], dstv)
    plsc.subcore_barrier()

    def issue(i, r1, r2, msg, sem):
        li = lax.rem(i, NH)
        pltpu.async_copy(e3_hbm.at[pl.ds(ebase + i * CH, CH)], msg, sem)
        pltpu.async_copy(p1_hbm.at[srcv.at[pl.ds(li * CH, CH)]], r1, sem)
        pltpu.async_copy(p2_hbm.at[dstv.at[li]], r2, sem)

    def drain(sem, n):
        for _ in range(n):
            pltpu.make_async_copy(e3_hbm.at[pl.ds(0, CH)], msga, sem).wait()

    def compute(r1, r2, msg):
        def row(r, carry):
            for k in range(D // 16):
                sl = pl.ds(k * 16, 16)
                msg[r, sl] = jnp.maximum(r1[r, sl] + r2[r, sl] + msg[r, sl], 0.0)
            return carry

        lax.fori_loop(0, CH, row, 0)

    issue(0, r1a, r2a, msga, sga)

    def body(i, carry):
        mi = lax.rem(i, NH)
        refill = jnp.logical_and(mi == NH - 1, i + 1 < NCHUNK)

        @pl.when(jnp.logical_not(refill))
        def _():
            @pl.when(i + 1 < NCHUNK)
            def _():
                @pl.when(i % 2 == 0)
                def _():
                    issue(i + 1, r1b, r2b, msgb, sgb)

                @pl.when(i % 2 == 1)
                def _():
                    issue(i + 1, r1a, r2a, msga, sga)

            @pl.when(i % 2 == 0)
            def _():
                drain(sga, 3)
                compute(r1a, r2a, msga)
                pltpu.async_copy(msga, accum.at[dstv.at[mi]], ssa, add=True)
                drain(ssa, 1)

            @pl.when(i % 2 == 1)
            def _():
                drain(sgb, 3)
                compute(r1b, r2b, msgb)
                pltpu.async_copy(msgb, accum.at[dstv.at[mi]], ssb, add=True)
                drain(ssb, 1)

        @pl.when(refill)
        def _():
            drain(sgb, 3)
            compute(r1b, r2b, msgb)
            pltpu.async_copy(msgb, accum.at[dstv.at[mi]], ssb, add=True)
            drain(ssb, 1)
            h = (i + 1) // NH
            pltpu.sync_copy(src_hbm.at[pl.ds(ebase + (i + 1) * CH, NH * CH)], srcv)
            pltpu.sync_copy(dst3_hbm.at[wid, h], dstv)
            issue(i + 1, r1a, r2a, msga, sga)

        return carry

    lax.fori_loop(0, NCHUNK, body, 0)
    plsc.subcore_barrier()

    rows = pl.ds(s * RPT, RPT)

    @pl.when(c == 0)
    def _():
        pltpu.sync_copy(accum.at[rows], out0_hbm.at[rows])

    @pl.when(c == 1)
    def _():
        pltpu.sync_copy(accum.at[rows], out1_hbm.at[rows])


@functools.cache
def _sc_layer_fn():
    return pl.kernel(
        _sc_body,
        out_type=[jax.ShapeDtypeStruct((NPAD, D), jnp.float32)] * 2,
        mesh=plsc.VectorSubcoreMesh(
            core_axis_name="c", subcore_axis_name="s", num_cores=2,
            num_subcores=16,
        ),
        scratch_types=[
            pltpu.VMEM_SHARED((NPAD, D), jnp.float32),
            pltpu.VMEM((NH * CH,), jnp.int32),
            pltpu.VMEM((NH, CH), jnp.int32),
            pltpu.VMEM((CH, D), jnp.float32),
            pltpu.VMEM((CH, D), jnp.float32),
            pltpu.VMEM((CH, D), jnp.float32),
            pltpu.VMEM((CH, D), jnp.float32),
            pltpu.VMEM((CH, D), jnp.float32),
            pltpu.VMEM((CH, D), jnp.float32),
            pltpu.SemaphoreType.DMA,
            pltpu.SemaphoreType.DMA,
            pltpu.SemaphoreType.DMA,
            pltpu.SemaphoreType.DMA,
        ],
    )


def _sc_layer(p1, p2, e3l, src, dst3, zero_nd):
    return _sc_layer_fn()(p1, p2, e3l, src, dst3, zero_nd)


def _upd_body(x_ref, a0_ref, a1_ref, wu_ref, bu_ref, wm_ref, xn_ref, p1_ref, p2_ref):
    x = x_ref[...]
    a = a0_ref[...] + a1_ref[...]
    u = (
        jnp.dot(x, wu_ref[:D], preferred_element_type=jnp.float32)
        + jnp.dot(a, wu_ref[D:], preferred_element_type=jnp.float32)
        + bu_ref[...]
    )
    xn = x + jnp.maximum(u, 0.0)
    xn_ref[...] = xn
    p1_ref[...] = jnp.dot(xn, wm_ref[:D], preferred_element_type=jnp.float32)
    p2_ref[...] = jnp.dot(xn, wm_ref[D:], preferred_element_type=jnp.float32)


def _update(x, a0, a1, wu, bu2d, wm12):
    return pl.pallas_call(
        _upd_body,
        grid=(NT,),
        in_specs=[
            pl.BlockSpec((TN, D), lambda t: (t, 0)),
            pl.BlockSpec((TN, D), lambda t: (t, 0)),
            pl.BlockSpec((TN, D), lambda t: (t, 0)),
            pl.BlockSpec((2 * D, D), lambda t: (0, 0)),
            pl.BlockSpec((1, D), lambda t: (0, 0)),
            pl.BlockSpec((2 * D, D), lambda t: (0, 0)),
        ],
        out_specs=[pl.BlockSpec((TN, D), lambda t: (t, 0))] * 3,
        out_shape=[jax.ShapeDtypeStruct((NPAD, D), jnp.float32)] * 3,
    )(x, a0, a1, wu, bu2d, wm12)


def _read_body(x_ref, b_ref, w1_ref, b1_ref, w2_ref, b2_ref, o_ref, pool_ref):
    t = pl.program_id(0)
    seg = lax.broadcasted_iota(jnp.int32, (TN, B), 1)
    oh = (b_ref[...] == seg).astype(jnp.float32)
    part = lax.dot_general(
        oh, x_ref[...], (((0,), (0,)), ((), ())),
        preferred_element_type=jnp.float32,
    )

    @pl.when(t == 0)
    def _():
        pool_ref[...] = part

    @pl.when(t > 0)
    def _():
        pool_ref[...] = pool_ref[...] + part

    @pl.when(t == NT - 1)
    def _():
        h = jnp.dot(pool_ref[...], w1_ref[...], preferred_element_type=jnp.float32)
        h = jnp.maximum(h + b1_ref[...], 0.0)
        o = jnp.sum(h * w2_ref[...], axis=1, keepdims=True) + b2_ref[...]
        o_ref[...] = o


def _readout(x, batch2d, out_W1, out_b12d, w2row, out_b22d):
    return pl.pallas_call(
        _read_body,
        grid=(NT,),
        in_specs=[
            pl.BlockSpec((TN, D), lambda t: (t, 0)),
            pl.BlockSpec((TN, 1), lambda t: (t, 0)),
            pl.BlockSpec((D, D), lambda t: (0, 0)),
            pl.BlockSpec((1, D), lambda t: (0, 0)),
            pl.BlockSpec((1, D), lambda t: (0, 0)),
            pl.BlockSpec((1, 1), lambda t: (0, 0)),
        ],
        out_specs=pl.BlockSpec((B, 1), lambda t: (0, 0)),
        out_shape=jax.ShapeDtypeStruct((B, 1), jnp.float32),
        scratch_shapes=[pltpu.VMEM((B, D), jnp.float32)],
    )(x, batch2d, out_W1, out_b12d, w2row, out_b22d)


def kernel(atomic_numbers, edge_index, edge_dist, batch, emb_table, proj_W,
           proj_b, edge_W, edge_b, Wm, bm, Wu, bu, out_W1, out_b1, out_W2,
           out_b2):
    f32 = jnp.float32
    an2d = jnp.zeros((NPAD, 1), jnp.int32).at[:N].set(
        atomic_numbers.astype(jnp.int32).reshape(N, 1))
    src = edge_index[0].astype(jnp.int32)
    dst3 = edge_index[1].astype(jnp.int32).reshape(NWORK, NCHUNK // NH, NH, CH)
    batch2d = jnp.full((NPAD, 1), B, jnp.int32).at[:N].set(
        batch.astype(jnp.int32).reshape(N, 1))

    tab = jnp.concatenate([emb_table, jnp.asarray(_PROPS)], axis=1)
    tab_pad = jnp.zeros((128, 132), f32).at[:119].set(tab)
    proj_b2d = proj_b.reshape(1, D)
    edge_b2d = edge_b.reshape(1, D)
    dist2d = edge_dist.reshape(E, 1)
    zero_nd = jnp.zeros((NPAD, D), f32)

    x, p1, p2 = _embed(an2d, tab_pad, proj_W, proj_b2d, Wm[0, : 2 * D])
    e3 = _eprep(dist2d, edge_W, edge_b2d, Wm[0, 2 * D :, :], bm[0].reshape(1, D))

    for l in range(L):
        a0, a1 = _sc_layer(p1, p2, e3, src, dst3, zero_nd)
        if l + 1 < L:
            e3 = _eprep(dist2d, edge_W, edge_b2d, Wm[l + 1, 2 * D :, :],
                        bm[l + 1].reshape(1, D))
        wm_next = Wm[(l + 1) % L, : 2 * D]
        x, p1, p2 = _update(x, a0, a1, Wu[l], bu[l].reshape(1, D), wm_next)

    return _readout(x, batch2d, out_W1, out_b1.reshape(1, D),
                    out_W2.reshape(1, D), out_b2.reshape(1, 1))


# speedup vs baseline: 6.0660x; 1.0013x over previous
import functools

import jax
import jax.numpy as jnp
import numpy as np
from jax import lax
from jax.experimental import pallas as pl
from jax.experimental.pallas import tpu as pltpu
from jax.experimental.pallas import tpu_sc as plsc

_MASSES = [0.0, 1.008, 4.003, 6.941, 9.012, 10.81, 12.01, 14.01, 16.0, 19.0, 20.18, 22.99, 24.31, 26.98, 28.09, 30.97, 32.07, 35.45, 39.95, 39.1, 40.08, 44.96, 47.87, 50.94, 52.0, 54.94, 55.85, 58.93, 58.69, 63.55, 65.38, 69.72, 72.63, 74.92, 78.97, 79.9, 83.8, 85.47, 87.62, 88.91, 91.22, 92.91, 95.95, 98.0, 101.1, 102.9, 106.4, 107.9, 112.4, 114.8, 118.7, 121.8, 127.6, 126.9, 131.3, 132.9, 137.3, 138.9, 140.1, 140.9, 144.2, 145.0, 150.4, 152.0, 157.3, 158.9, 162.5, 164.9, 167.3, 168.9, 173.0, 175.0, 178.5, 180.9, 183.8, 186.2, 190.2, 192.2, 195.1, 197.0, 200.6, 204.4, 207.2, 209.0, 209.0, 210.0, 222.0, 223.0, 226.0, 227.0, 232.0, 231.0, 238.0, 237.0, 244.0, 243.0, 247.0, 247.0, 251.0, 252.0, 257.0, 258.0, 259.0, 262.0, 267.0, 270.0, 269.0, 270.0, 270.0, 278.0, 281.0, 281.0, 285.0, 286.0, 289.0, 289.0, 293.0, 293.0, 294.0]
_RADII = [0.0, 1.2, 1.4, 1.82, 1.53, 1.92, 1.7, 1.55, 1.52, 1.47, 1.54, 2.27, 1.73, 1.84, 2.1, 1.8, 1.8, 1.75, 1.88, 2.75, 2.31, 2.11, 2.0, 2.0, 2.0, 2.0, 2.0, 2.0, 1.63, 1.4, 1.39, 1.87, 2.11, 1.85, 1.9, 1.85, 2.02, 3.03, 2.49, 2.0, 2.0, 2.0, 2.0, 2.0, 2.0, 2.0, 1.63, 1.72, 1.58, 1.93, 2.17, 2.06, 2.06, 1.98, 2.16, 3.43, 2.68, 2.0, 2.0, 2.0, 2.0, 2.0, 2.0, 2.0, 2.0, 2.0, 2.0, 2.0, 2.0, 2.0, 2.0, 2.0, 2.0, 2.0, 2.0, 2.0, 2.0, 2.0, 1.75, 1.66, 1.55, 1.96, 2.02, 2.07, 1.97, 2.02, 2.2, 3.48, 2.83, 2.0, 2.0, 2.0, 1.86, 2.0, 2.0, 2.0, 2.0, 2.0, 2.0, 2.0, 2.0, 2.0, 2.0, 2.0, 2.0, 2.0, 2.0, 2.0, 2.0, 2.0, 2.0, 2.0, 2.0, 2.0, 2.0, 2.0, 2.0, 2.0, 2.0]
_EN = [0.0, 2.2, 0.0, 0.98, 1.57, 2.04, 2.55, 3.04, 3.44, 3.98, 0.0, 0.93, 1.31, 1.61, 1.9, 2.19, 2.58, 3.16, 0.0, 0.82, 1.0, 1.36, 1.54, 1.63, 1.66, 1.55, 1.83, 1.88, 1.91, 1.9, 1.65, 1.81, 2.01, 2.18, 2.55, 2.96, 3.0, 0.82, 0.95, 1.22, 1.33, 1.6, 2.16, 1.9, 2.2, 2.28, 2.2, 1.93, 1.69, 1.78, 1.96, 2.05, 2.1, 2.66, 2.6, 0.79, 0.89, 1.1, 1.12, 1.13, 1.14, 1.13, 1.17, 1.2, 1.2, 1.22, 1.23, 1.24, 1.25, 1.1, 1.27, 1.3, 1.5, 2.36, 1.9, 2.2, 2.2, 2.28, 2.54, 2.0, 1.62, 1.87, 2.33, 2.02, 2.0, 2.2, 2.2, 0.7, 0.9, 1.1, 1.3, 1.5, 1.38, 1.36, 1.28, 1.3, 1.3, 1.3, 1.3, 1.3, 1.3, 1.3, 1.3, 1.3, 1.3, 1.3, 1.3, 1.3, 1.3, 1.3, 1.3, 1.3, 1.3, 1.3, 1.3, 1.3, 1.3, 1.3, 1.3]
_IE = [0.0, 13.6, 24.59, 5.39, 9.32, 8.3, 11.26, 14.53, 13.62, 17.42, 21.56, 5.14, 7.65, 5.99, 8.15, 10.49, 10.36, 12.97, 15.76, 4.34, 6.11, 6.56, 6.83, 6.75, 6.77, 7.43, 7.9, 7.88, 7.64, 7.73, 9.39, 6.0, 7.9, 9.79, 9.75, 11.81, 14.0, 4.18, 5.69, 6.22, 6.63, 6.76, 7.09, 7.28, 7.36, 7.46, 8.34, 7.58, 8.99, 5.79, 7.34, 8.64, 9.01, 10.45, 12.13, 3.89, 5.21, 5.58, 5.54, 5.47, 5.53, 5.58, 5.64, 5.67, 6.15, 5.86, 5.94, 6.02, 6.11, 6.18, 6.25, 5.43, 6.83, 7.55, 7.86, 7.83, 8.44, 8.97, 8.96, 9.23, 10.44, 6.11, 7.42, 7.29, 8.42, 9.3, 10.75, 4.07, 5.28, 5.17, 6.31, 5.89, 6.19, 6.27, 6.03, 5.97, 6.02, 6.2, 6.28, 6.42, 6.5, 6.58, 6.65, 4.9, 6.0, 6.0, 6.0, 6.0, 6.0, 6.0, 6.0, 6.0, 6.0, 6.0, 6.0, 6.0, 6.0, 6.0, 6.0]

_PROPS = np.zeros((119, 4), np.float32)
_PROPS[: len(_MASSES), 0] = _MASSES
_PROPS[: len(_RADII), 1] = _RADII
_PROPS[: len(_EN), 2] = _EN
_PROPS[: len(_IE), 3] = _IE

N = 10000
E = 320000
D = 128
NB = 64
L = 4
B = 64
CUT = 8.0

NT = 10
TN = 1024
ET = 160
TE = E // ET

NWORK = 32
EW = E // NWORK
CH = 40
NCHUNK = EW // CH
NPAD = 10240
RPT = NPAD // 16
NH = 50


def _embed_body(an_ref, tab_ref, pw_ref, pb_ref, wm_ref, x_ref, p1_ref, p2_ref):
    ft = jnp.dot(tab_ref[...], pw_ref[...], preferred_element_type=jnp.float32)
    an = jnp.clip(an_ref[...], 0, 118)
    lanes = lax.broadcasted_iota(jnp.int32, (TN, 128), 1)
    oh = (an == lanes).astype(jnp.float32)
    x = jnp.dot(oh, ft, preferred_element_type=jnp.float32) + pb_ref[...]
    x_ref[...] = x
    p1_ref[...] = jnp.dot(x, wm_ref[:D], preferred_element_type=jnp.float32)
    p2_ref[...] = jnp.dot(x, wm_ref[D:], preferred_element_type=jnp.float32)


def _embed(an2d, tab_pad, proj_W, proj_b2d, wm12):
    return pl.pallas_call(
        _embed_body,
        grid=(NT,),
        in_specs=[
            pl.BlockSpec((TN, 1), lambda t: (t, 0)),
            pl.BlockSpec((128, 132), lambda t: (0, 0)),
            pl.BlockSpec((132, D), lambda t: (0, 0)),
            pl.BlockSpec((1, D), lambda t: (0, 0)),
            pl.BlockSpec((2 * D, D), lambda t: (0, 0)),
        ],
        out_specs=[
            pl.BlockSpec((TN, D), lambda t: (t, 0)),
            pl.BlockSpec((TN, D), lambda t: (t, 0)),
            pl.BlockSpec((TN, D), lambda t: (t, 0)),
        ],
        out_shape=[jax.ShapeDtypeStruct((NPAD, D), jnp.float32)] * 3,
    )(an2d, tab_pad, proj_W, proj_b2d, wm12)


def _eprep_body(d_ref, ew_ref, eb_ref, w3_ref, bm_ref, o0, o1, o2, o3):
    d = d_ref[...]
    mu = lax.broadcasted_iota(jnp.int32, (TE, NB), 1).astype(jnp.float32) * (
        CUT / (NB - 1)
    )
    sigma = CUT / NB
    z = d - mu
    rbf = jnp.exp(z * z * (-1.0 / (2.0 * sigma * sigma)))
    e = jnp.dot(rbf, ew_ref[...], preferred_element_type=jnp.float32) + eb_ref[...]
    e = jnp.maximum(e, 0.0)
    for l, o in enumerate((o0, o1, o2, o3)):
        o[...] = (
            jnp.dot(e, w3_ref[l], preferred_element_type=jnp.float32)
            + bm_ref[l, :, :]
        )


def _eprep(dist2d, edge_W, edge_b2d, w3_all, bm3d):
    return pl.pallas_call(
        _eprep_body,
        grid=(ET,),
        in_specs=[
            pl.BlockSpec((TE, 1), lambda t: (t, 0)),
            pl.BlockSpec((NB, D), lambda t: (0, 0)),
            pl.BlockSpec((1, D), lambda t: (0, 0)),
            pl.BlockSpec((L, D, D), lambda t: (0, 0, 0)),
            pl.BlockSpec((L, 1, D), lambda t: (0, 0, 0)),
        ],
        out_specs=[pl.BlockSpec((TE, D), lambda t: (t, 0))] * L,
        out_shape=[jax.ShapeDtypeStruct((E, D), jnp.float32)] * L,
    )(dist2d, edge_W, edge_b2d, w3_all, bm3d)


def _sc_body(p1_hbm, p2_hbm, e3_hbm, src_hbm, dst3_hbm, zero_hbm,
             out0_hbm, out1_hbm,
             accum, srcv, dstv,
             r1a, r2a, msga, r1b, r2b, msgb,
             sga, sgb, ssa, ssb):
    c = lax.axis_index("c")
    s = lax.axis_index("s")
    wid = c * 16 + s
    ebase = wid * EW

    pltpu.sync_copy(zero_hbm.at[pl.ds(s * RPT, RPT)], accum.at[pl.ds(s * RPT, RPT)])
    pltpu.sync_copy(src_hbm.at[pl.ds(ebase, NH * CH)], srcv)
    pltpu.sync_copy(dst3_hbm.at[wid, 0], dstv)
    plsc.subcore_barrier()

    def issue(i, r1, r2, msg, sem):
        li = lax.rem(i, NH)
        pltpu.async_copy(e3_hbm.at[pl.ds(ebase + i * CH, CH)], msg, sem)
        pltpu.async_copy(p1_hbm.at[srcv.at[pl.ds(li * CH, CH)]], r1, sem)
        pltpu.async_copy(p2_hbm.at[dstv.at[li]], r2, sem)

    def drain(sem, n):
        for _ in range(n):
            pltpu.make_async_copy(e3_hbm.at[pl.ds(0, CH)], msga, sem).wait()

    def compute(r1, r2, msg):
        def row(r, carry):
            for k in range(D // 16):
                sl = pl.ds(k * 16, 16)
                msg[r, sl] = jnp.maximum(r1[r, sl] + r2[r, sl] + msg[r, sl], 0.0)
            return carry

        lax.fori_loop(0, CH, row, 0)

    issue(0, r1a, r2a, msga, sga)

    def body(i, carry):
        mi = lax.rem(i, NH)
        refill = jnp.logical_and(mi == NH - 1, i + 1 < NCHUNK)

        @pl.when(jnp.logical_not(refill))
        def _():
            @pl.when(i + 1 < NCHUNK)
            def _():
                @pl.when(i % 2 == 0)
                def _():
                    issue(i + 1, r1b, r2b, msgb, sgb)

                @pl.when(i % 2 == 1)
                def _():
                    issue(i + 1, r1a, r2a, msga, sga)

            @pl.when(i % 2 == 0)
            def _():
                drain(sga, 3)
                compute(r1a, r2a, msga)
                pltpu.async_copy(msga, accum.at[dstv.at[mi]], ssa, add=True)
                drain(ssa, 1)

            @pl.when(i % 2 == 1)
            def _():
                drain(sgb, 3)
                compute(r1b, r2b, msgb)
                pltpu.async_copy(msgb, accum.at[dstv.at[mi]], ssb, add=True)
                drain(ssb, 1)

        @pl.when(refill)
        def _():
            drain(sgb, 3)
            compute(r1b, r2b, msgb)
            pltpu.async_copy(msgb, accum.at[dstv.at[mi]], ssb, add=True)
            drain(ssb, 1)
            h = (i + 1) // NH
            pltpu.sync_copy(src_hbm.at[pl.ds(ebase + (i + 1) * CH, NH * CH)], srcv)
            pltpu.sync_copy(dst3_hbm.at[wid, h], dstv)
            issue(i + 1, r1a, r2a, msga, sga)

        return carry

    lax.fori_loop(0, NCHUNK, body, 0)
    plsc.subcore_barrier()

    rows = pl.ds(s * RPT, RPT)

    @pl.when(c == 0)
    def _():
        pltpu.sync_copy(accum.at[rows], out0_hbm.at[rows])

    @pl.when(c == 1)
    def _():
        pltpu.sync_copy(accum.at[rows], out1_hbm.at[rows])


@functools.cache
def _sc_layer_fn():
    return pl.kernel(
        _sc_body,
        out_type=[jax.ShapeDtypeStruct((NPAD, D), jnp.float32)] * 2,
        mesh=plsc.VectorSubcoreMesh(
            core_axis_name="c", subcore_axis_name="s", num_cores=2,
            num_subcores=16,
        ),
        scratch_types=[
            pltpu.VMEM_SHARED((NPAD, D), jnp.float32),
            pltpu.VMEM((NH * CH,), jnp.int32),
            pltpu.VMEM((NH, CH), jnp.int32),
            pltpu.VMEM((CH, D), jnp.float32),
            pltpu.VMEM((CH, D), jnp.float32),
            pltpu.VMEM((CH, D), jnp.float32),
            pltpu.VMEM((CH, D), jnp.float32),
            pltpu.VMEM((CH, D), jnp.float32),
            pltpu.VMEM((CH, D), jnp.float32),
            pltpu.SemaphoreType.DMA,
            pltpu.SemaphoreType.DMA,
            pltpu.SemaphoreType.DMA,
            pltpu.SemaphoreType.DMA,
        ],
    )


def _sc_layer(p1, p2, e3l, src, dst3, zero_nd):
    return _sc_layer_fn()(p1, p2, e3l, src, dst3, zero_nd)


def _upd_body(x_ref, a0_ref, a1_ref, wu_ref, bu_ref, wm_ref, xn_ref, p1_ref, p2_ref):
    x = x_ref[...]
    a = a0_ref[...] + a1_ref[...]
    u = (
        jnp.dot(x, wu_ref[:D], preferred_element_type=jnp.float32)
        + jnp.dot(a, wu_ref[D:], preferred_element_type=jnp.float32)
        + bu_ref[...]
    )
    xn = x + jnp.maximum(u, 0.0)
    xn_ref[...] = xn
    p1_ref[...] = jnp.dot(xn, wm_ref[:D], preferred_element_type=jnp.float32)
    p2_ref[...] = jnp.dot(xn, wm_ref[D:], preferred_element_type=jnp.float32)


def _update(x, a0, a1, wu, bu2d, wm12):
    return pl.pallas_call(
        _upd_body,
        grid=(NT,),
        in_specs=[
            pl.BlockSpec((TN, D), lambda t: (t, 0)),
            pl.BlockSpec((TN, D), lambda t: (t, 0)),
            pl.BlockSpec((TN, D), lambda t: (t, 0)),
            pl.BlockSpec((2 * D, D), lambda t: (0, 0)),
            pl.BlockSpec((1, D), lambda t: (0, 0)),
            pl.BlockSpec((2 * D, D), lambda t: (0, 0)),
        ],
        out_specs=[pl.BlockSpec((TN, D), lambda t: (t, 0))] * 3,
        out_shape=[jax.ShapeDtypeStruct((NPAD, D), jnp.float32)] * 3,
    )(x, a0, a1, wu, bu2d, wm12)


def _read_body(x_ref, b_ref, w1_ref, b1_ref, w2_ref, b2_ref, o_ref, pool_ref):
    t = pl.program_id(0)
    seg = lax.broadcasted_iota(jnp.int32, (TN, B), 1)
    oh = (b_ref[...] == seg).astype(jnp.float32)
    part = lax.dot_general(
        oh, x_ref[...], (((0,), (0,)), ((), ())),
        preferred_element_type=jnp.float32,
    )

    @pl.when(t == 0)
    def _():
        pool_ref[...] = part

    @pl.when(t > 0)
    def _():
        pool_ref[...] = pool_ref[...] + part

    @pl.when(t == NT - 1)
    def _():
        h = jnp.dot(pool_ref[...], w1_ref[...], preferred_element_type=jnp.float32)
        h = jnp.maximum(h + b1_ref[...], 0.0)
        o = jnp.sum(h * w2_ref[...], axis=1, keepdims=True) + b2_ref[...]
        o_ref[...] = o


def _readout(x, batch2d, out_W1, out_b12d, w2row, out_b22d):
    return pl.pallas_call(
        _read_body,
        grid=(NT,),
        in_specs=[
            pl.BlockSpec((TN, D), lambda t: (t, 0)),
            pl.BlockSpec((TN, 1), lambda t: (t, 0)),
            pl.BlockSpec((D, D), lambda t: (0, 0)),
            pl.BlockSpec((1, D), lambda t: (0, 0)),
            pl.BlockSpec((1, D), lambda t: (0, 0)),
            pl.BlockSpec((1, 1), lambda t: (0, 0)),
        ],
        out_specs=pl.BlockSpec((B, 1), lambda t: (0, 0)),
        out_shape=jax.ShapeDtypeStruct((B, 1), jnp.float32),
        scratch_shapes=[pltpu.VMEM((B, D), jnp.float32)],
    )(x, batch2d, out_W1, out_b12d, w2row, out_b22d)


def kernel(atomic_numbers, edge_index, edge_dist, batch, emb_table, proj_W,
           proj_b, edge_W, edge_b, Wm, bm, Wu, bu, out_W1, out_b1, out_W2,
           out_b2):
    f32 = jnp.float32
    an2d = jnp.zeros((NPAD, 1), jnp.int32).at[:N].set(
        atomic_numbers.astype(jnp.int32).reshape(N, 1))
    src = edge_index[0].astype(jnp.int32)
    dst3 = edge_index[1].astype(jnp.int32).reshape(NWORK, NCHUNK // NH, NH, CH)
    batch2d = jnp.full((NPAD, 1), B, jnp.int32).at[:N].set(
        batch.astype(jnp.int32).reshape(N, 1))

    tab = jnp.concatenate([emb_table, jnp.asarray(_PROPS)], axis=1)
    tab_pad = jnp.zeros((128, 132), f32).at[:119].set(tab)
    proj_b2d = proj_b.reshape(1, D)
    edge_b2d = edge_b.reshape(1, D)
    dist2d = edge_dist.reshape(E, 1)
    zero_nd = jnp.zeros((NPAD, D), f32)

    x, p1, p2 = _embed(an2d, tab_pad, proj_W, proj_b2d, Wm[0, : 2 * D])
    e3 = _eprep(dist2d, edge_W, edge_b2d, Wm[:, 2 * D :, :], bm.reshape(L, 1, D))

    for l in range(L):
        a0, a1 = _sc_layer(p1, p2, e3[l], src, dst3, zero_nd)
        wm_next = Wm[(l + 1) % L, : 2 * D]
        x, p1, p2 = _update(x, a0, a1, Wu[l], bu[l].reshape(1, D), wm_next)

    return _readout(x, batch2d, out_W1, out_b1.reshape(1, D),
                    out_W2.reshape(1, D), out_b2.reshape(1, 1))
